# Initial kernel scaffold; baseline (speedup 1.0000x reference)
import jax
import jax.numpy as jnp
from jax.experimental import pallas as pl


def kernel(src_emb, src_emb_in, dst_ids):
    raise NotImplementedError("write your pallas kernel here")


# speedup vs baseline: 4.9306x; 4.9306x over previous
import functools

import jax
import jax.numpy as jnp
from jax import lax
from jax.experimental import pallas as pl
from jax.experimental.pallas import tpu as pltpu
from jax.experimental.pallas import tpu_sc as plsc

N_DST = 10000
N_EDGES = 320000
D = 128

NC = 2
NS = 16
NW = NC * NS

EW = N_EDGES // NW
C = 80
K = EW // C

SPT = 624
REM = N_DST - NS * SPT


def _sc_partials(src_emb, idx3, zeros):
  mesh = plsc.VectorSubcoreMesh(core_axis_name="c", subcore_axis_name="s")

  @functools.partial(
      pl.kernel,
      mesh=mesh,
      out_type=jax.ShapeDtypeStruct((NC, N_DST, D), jnp.float32),
      scratch_types=[
          pltpu.VMEM((K, C), jnp.int32),
          pltpu.VMEM((C, D), jnp.float32),
          pltpu.VMEM_SHARED((N_DST, D), jnp.float32),
      ],
  )
  def body(src_hbm, idx_hbm, zeros_hbm, out_hbm, idx_v, rows_v, acc_sh):
    c = lax.axis_index("c")
    s = lax.axis_index("s")
    wid = c * NS + s

    @pl.when(jnp.logical_and(s == 0, c == 0))
    def _():
      pltpu.sync_copy(src_hbm.at[pl.ds(N_EDGES, N_DST)], acc_sh)

    @pl.when(jnp.logical_and(s == 0, c == 1))
    def _():
      pltpu.sync_copy(zeros_hbm, acc_sh)

    plsc.subcore_barrier()

    pltpu.sync_copy(idx_hbm.at[wid], idx_v)

    base_e = wid * EW

    def chunk(j, carry):
      pltpu.sync_copy(src_hbm.at[pl.ds(base_e + j * C, C)], rows_v)
      pltpu.sync_copy(rows_v, acc_sh.at[idx_v.at[j]], add=True)
      return carry

    lax.fori_loop(0, K, chunk, 0)

    plsc.subcore_barrier()

    r0 = s * SPT
    pltpu.sync_copy(acc_sh.at[pl.ds(r0, SPT)],
                    out_hbm.at[c, pl.ds(r0, SPT)])

    @pl.when(s == NS - 1)
    def _():
      pltpu.sync_copy(acc_sh.at[pl.ds(NS * SPT, REM)],
                      out_hbm.at[c, pl.ds(NS * SPT, REM)])

  return body(src_emb, idx3, zeros)


def _final_add(partials):
  def body(a_ref, o_ref):
    o_ref[...] = a_ref[0] + a_ref[1]

  return pl.pallas_call(
      body,
      grid=(10,),
      in_specs=[pl.BlockSpec((NC, 1000, D), lambda i: (0, i, 0))],
      out_specs=pl.BlockSpec((1000, D), lambda i: (i, 0)),
      out_shape=jax.ShapeDtypeStruct((N_DST, D), jnp.float32),
  )(partials)


def kernel(src_emb, src_emb_in, dst_ids):
  del src_emb_in
  idx3 = dst_ids.astype(jnp.int32).reshape(NW, K, C)
  zeros = jnp.zeros((N_DST, D), jnp.float32)
  partials = _sc_partials(src_emb, idx3, zeros)
  return _final_add(partials)


# speedup vs baseline: 7.9224x; 1.6068x over previous
import functools

import jax
import jax.numpy as jnp
from jax import lax
from jax.experimental import pallas as pl
from jax.experimental.pallas import tpu as pltpu
from jax.experimental.pallas import tpu_sc as plsc

N_DST = 10000
N_EDGES = 320000
D = 128

NC = 2
NS = 16
NW = NC * NS

EW = N_EDGES // NW
C = 80
K = EW // C

SPT = 624
REM = N_DST - NS * SPT


def _sc_partials(src_emb, idx3, zeros):
  mesh = plsc.VectorSubcoreMesh(core_axis_name="c", subcore_axis_name="s")

  @functools.partial(
      pl.kernel,
      mesh=mesh,
      out_type=jax.ShapeDtypeStruct((NC, N_DST, D), jnp.float32),
      scratch_types=[
          pltpu.VMEM((K, C), jnp.int32),
          pltpu.VMEM((C, D), jnp.float32),
          pltpu.VMEM((C, D), jnp.float32),
          pltpu.VMEM_SHARED((N_DST, D), jnp.float32),
          pltpu.SemaphoreType.DMA,
          pltpu.SemaphoreType.DMA,
      ],
  )
  def body(src_hbm, idx_hbm, zeros_hbm, out_hbm, idx_v, rows0, rows1,
           acc_sh, sem0, sem1):
    c = lax.axis_index("c")
    s = lax.axis_index("s")
    wid = c * NS + s
    base_e = wid * EW

    def gather(j, buf, sem):
      return pltpu.async_copy(src_hbm.at[pl.ds(base_e + j * C, C)], buf, sem)

    def gather_wait(j, buf, sem):
      pltpu.make_async_copy(src_hbm.at[pl.ds(base_e + j * C, C)], buf,
                            sem).wait()

    def scatter(j, buf):
      pltpu.sync_copy(buf, acc_sh.at[idx_v.at[j]], add=True)

    pltpu.sync_copy(idx_hbm.at[wid], idx_v)
    gather(0, rows0, sem0)

    @pl.when(jnp.logical_and(s == 0, c == 0))
    def _():
      pltpu.sync_copy(src_hbm.at[pl.ds(N_EDGES, N_DST)], acc_sh)

    @pl.when(jnp.logical_and(s == 0, c == 1))
    def _():
      pltpu.sync_copy(zeros_hbm, acc_sh)

    plsc.subcore_barrier()

    def pair(i, carry):
      j = 2 * i
      gather(j + 1, rows1, sem1)
      gather_wait(j, rows0, sem0)
      scatter(j, rows0)
      gather(j + 2, rows0, sem0)
      gather_wait(j + 1, rows1, sem1)
      scatter(j + 1, rows1)
      return carry

    lax.fori_loop(0, (K - 1) // 2, pair, 0)
    gather_wait(K - 1, rows0, sem0)
    scatter(K - 1, rows0)

    plsc.subcore_barrier()

    r0 = s * SPT
    pltpu.sync_copy(acc_sh.at[pl.ds(r0, SPT)],
                    out_hbm.at[c, pl.ds(r0, SPT)])

    @pl.when(s == NS - 1)
    def _():
      pltpu.sync_copy(acc_sh.at[pl.ds(NS * SPT, REM)],
                      out_hbm.at[c, pl.ds(NS * SPT, REM)])

  return body(src_emb, idx3, zeros)


def _final_add(partials):
  def body(a_ref, o_ref):
    o_ref[...] = a_ref[0] + a_ref[1]

  return pl.pallas_call(
      body,
      grid=(10,),
      in_specs=[pl.BlockSpec((NC, 1000, D), lambda i: (0, i, 0))],
      out_specs=pl.BlockSpec((1000, D), lambda i: (i, 0)),
      out_shape=jax.ShapeDtypeStruct((N_DST, D), jnp.float32),
  )(partials)


def kernel(src_emb, src_emb_in, dst_ids):
  del src_emb_in
  idx3 = dst_ids.astype(jnp.int32).reshape(NW, K, C)
  zeros = jnp.zeros((N_DST, D), jnp.float32)
  partials = _sc_partials(src_emb, idx3, zeros)
  return _final_add(partials)


# speedup vs baseline: 8.9936x; 1.1352x over previous
import functools

import jax
import jax.numpy as jnp
from jax import lax
from jax.experimental import pallas as pl
from jax.experimental.pallas import tpu as pltpu
from jax.experimental.pallas import tpu_sc as plsc

N_DST = 10000
N_EDGES = 320000
D = 128

NC = 2
NS = 16
NW = NC * NS

EW = N_EDGES // NW
C = 80
K = EW // C

SPT = 624
REM = N_DST - NS * SPT

NB = 3
S = 2
G = NB - S


def _sc_partials(src_emb, idx3, zeros):
  mesh = plsc.VectorSubcoreMesh(core_axis_name="c", subcore_axis_name="s")

  @functools.partial(
      pl.kernel,
      mesh=mesh,
      out_type=jax.ShapeDtypeStruct((NC, N_DST, D), jnp.float32),
      scratch_types=[
          pltpu.VMEM((K, C), jnp.int32),
          pltpu.VMEM((NB, C, D), jnp.float32),
          pltpu.VMEM_SHARED((N_DST, D), jnp.float32),
          [pltpu.SemaphoreType.DMA] * NB,
          [pltpu.SemaphoreType.DMA] * NB,
      ],
  )
  def body(src_hbm, idx_hbm, zeros_hbm, out_hbm, idx_v, rows, acc_sh,
           gsems, ssems):
    c = lax.axis_index("c")
    s = lax.axis_index("s")
    wid = c * NS + s
    base_e = wid * EW

    def gslice(j):
      return src_hbm.at[pl.ds(base_e + j * C, C)]

    def gather_start(j, b):
      pltpu.async_copy(gslice(j), rows.at[b], gsems[b])

    def gather_wait(j, b):
      pltpu.make_async_copy(gslice(j), rows.at[b], gsems[b]).wait()

    def scatter_start(j, b):
      pltpu.async_copy(rows.at[b], acc_sh.at[idx_v.at[j]], ssems[b],
                       add=True)

    def scatter_wait(j, b):
      pltpu.make_async_copy(rows.at[b], acc_sh.at[idx_v.at[j]],
                            ssems[b]).wait()

    pltpu.sync_copy(idx_hbm.at[wid], idx_v)
    for b in range(G):
      gather_start(b, b)

    r0 = s * SPT

    @pl.when(c == 0)
    def _():
      pltpu.sync_copy(src_hbm.at[pl.ds(N_EDGES + r0, SPT)],
                      acc_sh.at[pl.ds(r0, SPT)])

      @pl.when(s == NS - 1)
      def _():
        pltpu.sync_copy(src_hbm.at[pl.ds(N_EDGES + NS * SPT, REM)],
                        acc_sh.at[pl.ds(NS * SPT, REM)])

    @pl.when(c == 1)
    def _():
      pltpu.sync_copy(zeros_hbm.at[pl.ds(r0, SPT)],
                      acc_sh.at[pl.ds(r0, SPT)])

      @pl.when(s == NS - 1)
      def _():
        pltpu.sync_copy(zeros_hbm.at[pl.ds(NS * SPT, REM)],
                        acc_sh.at[pl.ds(NS * SPT, REM)])

    plsc.subcore_barrier()

    for b in range(NB):
      if b >= S:
        scatter_wait(b - S, b - S)
      gather_start(b + G, (b + G) % NB)
      gather_wait(b, b)
      scatter_start(b, b)

    def group(i, carry):
      for b in range(NB):
        j = NB * i + b
        scatter_wait(j - S, (b - S) % NB)
        gather_start(j + G, (b + G) % NB)
        gather_wait(j, b)
        scatter_start(j, b)
      return carry

    lax.fori_loop(1, K // NB - 1, group, 0)

    tail = (K // NB - 1) * NB
    for t in range(tail, K):
      b = t % NB
      scatter_wait(t - S, (b - S) % NB)
      if t + G < K:
        gather_start(t + G, (t + G) % NB)
      gather_wait(t, b)
      scatter_start(t, b)
    for t in range(K - S, K):
      scatter_wait(t, t % NB)

    plsc.subcore_barrier()

    r0 = s * SPT
    pltpu.sync_copy(acc_sh.at[pl.ds(r0, SPT)],
                    out_hbm.at[c, pl.ds(r0, SPT)])

    @pl.when(s == NS - 1)
    def _():
      pltpu.sync_copy(acc_sh.at[pl.ds(NS * SPT, REM)],
                      out_hbm.at[c, pl.ds(NS * SPT, REM)])

  return body(src_emb, idx3, zeros)


def _final_add(partials):
  def body(a_ref, o_ref):
    o_ref[...] = a_ref[0] + a_ref[1]

  return pl.pallas_call(
      body,
      grid=(10,),
      in_specs=[pl.BlockSpec((NC, 1000, D), lambda i: (0, i, 0))],
      out_specs=pl.BlockSpec((1000, D), lambda i: (i, 0)),
      out_shape=jax.ShapeDtypeStruct((N_DST, D), jnp.float32),
  )(partials)


def kernel(src_emb, src_emb_in, dst_ids):
  del src_emb_in
  idx3 = dst_ids.astype(jnp.int32).reshape(NW, K, C)
  zeros = jnp.zeros((N_DST, D), jnp.float32)
  partials = _sc_partials(src_emb, idx3, zeros)
  return _final_add(partials)


# speedup vs baseline: 9.0942x; 1.0112x over previous
import functools

import jax
import jax.numpy as jnp
from jax import lax
from jax.experimental import pallas as pl
from jax.experimental.pallas import tpu as pltpu
from jax.experimental.pallas import tpu_sc as plsc

N_DST = 10000
N_EDGES = 320000
D = 128

NC = 2
NS = 16
NW = NC * NS

EW = N_EDGES // NW
C = 80
K = EW // C

SPT = 624
REM = N_DST - NS * SPT

NB = 4
S = 2
G = NB - S


def _sc_partials(src_emb, dst_ids, zeros):
  mesh = plsc.VectorSubcoreMesh(core_axis_name="c", subcore_axis_name="s")

  @functools.partial(
      pl.kernel,
      mesh=mesh,
      out_type=jax.ShapeDtypeStruct((NC, N_DST, D), jnp.float32),
      scratch_types=[
          pltpu.VMEM((EW,), jnp.int32),
          pltpu.VMEM((NB, C, D), jnp.float32),
          pltpu.VMEM_SHARED((N_DST, D), jnp.float32),
          [pltpu.SemaphoreType.DMA] * NB,
          [pltpu.SemaphoreType.DMA] * NB,
      ],
  )
  def body(src_hbm, idx_hbm, zeros_hbm, out_hbm, idx_v, rows, acc_sh,
           gsems, ssems):
    c = lax.axis_index("c")
    s = lax.axis_index("s")
    wid = c * NS + s
    base_e = wid * EW

    def gslice(j):
      return src_hbm.at[pl.ds(base_e + j * C, C)]

    def gather_start(j, b):
      pltpu.async_copy(gslice(j), rows.at[b], gsems[b])

    def gather_wait(j, b):
      pltpu.make_async_copy(gslice(j), rows.at[b], gsems[b]).wait()

    def scatter_start(j, b):
      pltpu.async_copy(rows.at[b], acc_sh.at[idx_v.at[pl.ds(j * C, C)]],
                       ssems[b], add=True)

    def scatter_wait(j, b):
      pltpu.make_async_copy(rows.at[b], acc_sh.at[idx_v.at[pl.ds(j * C, C)]],
                            ssems[b]).wait()

    pltpu.sync_copy(idx_hbm.at[pl.ds(base_e, EW)], idx_v)
    for b in range(G):
      gather_start(b, b)

    r0 = s * SPT

    @pl.when(c == 0)
    def _():
      pltpu.sync_copy(src_hbm.at[pl.ds(N_EDGES + r0, SPT)],
                      acc_sh.at[pl.ds(r0, SPT)])

      @pl.when(s == NS - 1)
      def _():
        pltpu.sync_copy(src_hbm.at[pl.ds(N_EDGES + NS * SPT, REM)],
                        acc_sh.at[pl.ds(NS * SPT, REM)])

    @pl.when(c == 1)
    def _():
      pltpu.sync_copy(zeros_hbm.at[pl.ds(0, SPT)],
                      acc_sh.at[pl.ds(r0, SPT)])

      @pl.when(s == NS - 1)
      def _():
        pltpu.sync_copy(zeros_hbm.at[pl.ds(0, REM)],
                        acc_sh.at[pl.ds(NS * SPT, REM)])

    plsc.subcore_barrier()

    for b in range(NB):
      if b >= S:
        scatter_wait(b - S, b - S)
      gather_start(b + G, (b + G) % NB)
      gather_wait(b, b)
      scatter_start(b, b)

    def group(i, carry):
      for b in range(NB):
        j = NB * i + b
        scatter_wait(j - S, (b - S) % NB)
        gather_start(j + G, (b + G) % NB)
        gather_wait(j, b)
        scatter_start(j, b)
      return carry

    lax.fori_loop(1, K // NB - 1, group, 0)

    tail = (K // NB - 1) * NB
    for t in range(tail, K):
      b = t % NB
      scatter_wait(t - S, (b - S) % NB)
      if t + G < K:
        gather_start(t + G, (t + G) % NB)
      gather_wait(t, b)
      scatter_start(t, b)
    for t in range(K - S, K):
      scatter_wait(t, t % NB)

    plsc.subcore_barrier()

    r0 = s * SPT
    pltpu.sync_copy(acc_sh.at[pl.ds(r0, SPT)],
                    out_hbm.at[c, pl.ds(r0, SPT)])

    @pl.when(s == NS - 1)
    def _():
      pltpu.sync_copy(acc_sh.at[pl.ds(NS * SPT, REM)],
                      out_hbm.at[c, pl.ds(NS * SPT, REM)])

  return body(src_emb, dst_ids, zeros)


def _final_add(partials):
  def body(a_ref, o_ref):
    o_ref[...] = a_ref[0] + a_ref[1]

  return pl.pallas_call(
      body,
      grid=(5,),
      in_specs=[pl.BlockSpec((NC, 2000, D), lambda i: (0, i, 0))],
      out_specs=pl.BlockSpec((2000, D), lambda i: (i, 0)),
      out_shape=jax.ShapeDtypeStruct((N_DST, D), jnp.float32),
  )(partials)


ZR = SPT + REM


def kernel(src_emb, src_emb_in, dst_ids):
  del src_emb_in
  zeros = jnp.zeros((ZR, D), jnp.float32)
  partials = _sc_partials(src_emb, dst_ids.astype(jnp.int32), zeros)
  return _final_add(partials)


# speedup vs baseline: 9.4037x; 1.0340x over previous
import functools

import jax
import jax.numpy as jnp
from jax import lax
from jax.experimental import pallas as pl
from jax.experimental.pallas import tpu as pltpu
from jax.experimental.pallas import tpu_sc as plsc

N_DST = 10000
N_EDGES = 320000
D = 128

NC = 2
NS = 16
NW = NC * NS

EW = N_EDGES // NW
C = 80
K = EW // C

SPT = 624
REM = N_DST - NS * SPT

NB = 3
S = 2
G = NB - S


def _sc_partials(src_emb, dst_ids, zeros):
  mesh = plsc.VectorSubcoreMesh(core_axis_name="c", subcore_axis_name="s")

  @functools.partial(
      pl.kernel,
      mesh=mesh,
      out_type=jax.ShapeDtypeStruct((NC, N_DST, D), jnp.float32),
      scratch_types=[
          pltpu.VMEM((EW,), jnp.int32),
          pltpu.VMEM((NB, C, D), jnp.float32),
          pltpu.VMEM_SHARED((N_DST, D), jnp.float32),
          [pltpu.SemaphoreType.DMA] * NB,
          [pltpu.SemaphoreType.DMA] * NB,
      ],
  )
  def body(src_hbm, idx_hbm, zeros_hbm, out_hbm, idx_v, rows, acc_sh,
           gsems, ssems):
    c = lax.axis_index("c")
    s = lax.axis_index("s")
    wid = c * NS + s
    base_e = wid * EW

    def gslice(j):
      return src_hbm.at[pl.ds(base_e + j * C, C)]

    def gather_start(j, b):
      pltpu.async_copy(gslice(j), rows.at[b], gsems[b])

    def gather_wait(j, b):
      pltpu.make_async_copy(gslice(j), rows.at[b], gsems[b]).wait()

    def scatter_start(j, b):
      pltpu.async_copy(rows.at[b], acc_sh.at[idx_v.at[pl.ds(j * C, C)]],
                       ssems[b], add=True)

    def scatter_wait(j, b):
      pltpu.make_async_copy(rows.at[b], acc_sh.at[idx_v.at[pl.ds(j * C, C)]],
                            ssems[b]).wait()

    pltpu.sync_copy(idx_hbm.at[pl.ds(base_e, EW)], idx_v)
    for b in range(G):
      gather_start(b, b)

    r0 = s * SPT

    @pl.when(c == 0)
    def _():
      pltpu.sync_copy(src_hbm.at[pl.ds(N_EDGES + r0, SPT)],
                      acc_sh.at[pl.ds(r0, SPT)])

      @pl.when(s == NS - 1)
      def _():
        pltpu.sync_copy(src_hbm.at[pl.ds(N_EDGES + NS * SPT, REM)],
                        acc_sh.at[pl.ds(NS * SPT, REM)])

    @pl.when(c == 1)
    def _():
      pltpu.sync_copy(zeros_hbm.at[pl.ds(0, SPT)],
                      acc_sh.at[pl.ds(r0, SPT)])

      @pl.when(s == NS - 1)
      def _():
        pltpu.sync_copy(zeros_hbm.at[pl.ds(0, REM)],
                        acc_sh.at[pl.ds(NS * SPT, REM)])

    plsc.subcore_barrier()

    for b in range(NB):
      if b >= S:
        scatter_wait(b - S, b - S)
      gather_start(b + G, (b + G) % NB)
      gather_wait(b, b)
      scatter_start(b, b)

    def group(i, carry):
      for b in range(NB):
        j = NB * i + b
        scatter_wait(j - S, (b - S) % NB)
        gather_start(j + G, (b + G) % NB)
        gather_wait(j, b)
        scatter_start(j, b)
      return carry

    lax.fori_loop(1, K // NB - 1, group, 0)

    tail = (K // NB - 1) * NB
    for t in range(tail, K):
      b = t % NB
      scatter_wait(t - S, (b - S) % NB)
      if t + G < K:
        gather_start(t + G, (t + G) % NB)
      gather_wait(t, b)
      scatter_start(t, b)
    for t in range(K - S, K):
      scatter_wait(t, t % NB)

    plsc.subcore_barrier()

    r0 = s * SPT
    pltpu.sync_copy(acc_sh.at[pl.ds(r0, SPT)],
                    out_hbm.at[c, pl.ds(r0, SPT)])

    @pl.when(s == NS - 1)
    def _():
      pltpu.sync_copy(acc_sh.at[pl.ds(NS * SPT, REM)],
                      out_hbm.at[c, pl.ds(NS * SPT, REM)])

  return body(src_emb, dst_ids, zeros)


def _final_add(partials):
  def body(a_ref, o_ref):
    o_ref[...] = a_ref[0] + a_ref[1]

  return pl.pallas_call(
      body,
      grid=(5,),
      in_specs=[pl.BlockSpec((NC, 2000, D), lambda i: (0, i, 0))],
      out_specs=pl.BlockSpec((2000, D), lambda i: (i, 0)),
      out_shape=jax.ShapeDtypeStruct((N_DST, D), jnp.float32),
  )(partials)


ZR = SPT + REM


def kernel(src_emb, src_emb_in, dst_ids):
  del src_emb_in
  zeros = jnp.zeros((ZR, D), jnp.float32)
  partials = _sc_partials(src_emb, dst_ids.astype(jnp.int32), zeros)
  return _final_add(partials)


# speedup vs baseline: 9.9938x; 1.0628x over previous
import functools

import jax
import jax.numpy as jnp
from jax import lax
from jax.experimental import pallas as pl
from jax.experimental.pallas import tpu as pltpu
from jax.experimental.pallas import tpu_sc as plsc

N_DST = 10000
N_EDGES = 320000
D = 128

NC = 2
NS = 16
NW = NC * NS

EW = N_EDGES // NW
C = 80
K = EW // C

SPT = 624
REM = N_DST - NS * SPT

NB = 4
S = 1
G = NB - S


def _sc_partials(src_emb, dst_ids, zeros):
  mesh = plsc.VectorSubcoreMesh(core_axis_name="c", subcore_axis_name="s")

  @functools.partial(
      pl.kernel,
      mesh=mesh,
      out_type=jax.ShapeDtypeStruct((NC, N_DST, D), jnp.float32),
      scratch_types=[
          pltpu.VMEM((EW,), jnp.int32),
          pltpu.VMEM((NB, C, D), jnp.float32),
          pltpu.VMEM_SHARED((N_DST, D), jnp.float32),
          [pltpu.SemaphoreType.DMA] * NB,
          [pltpu.SemaphoreType.DMA] * NB,
          pltpu.SemaphoreType.DMA,
      ],
  )
  def body(src_hbm, idx_hbm, zeros_hbm, out_hbm, idx_v, rows, acc_sh,
           gsems, ssems, isem):
    c = lax.axis_index("c")
    s = lax.axis_index("s")
    wid = c * NS + s
    base_e = wid * EW

    def gslice(j):
      return src_hbm.at[pl.ds(base_e + j * C, C)]

    def gather_start(j, b):
      pltpu.async_copy(gslice(j), rows.at[b], gsems[b])

    def gather_wait(j, b):
      pltpu.make_async_copy(gslice(j), rows.at[b], gsems[b]).wait()

    def scatter_start(j, b):
      pltpu.async_copy(rows.at[b], acc_sh.at[idx_v.at[pl.ds(j * C, C)]],
                       ssems[b], add=True)

    def scatter_wait(j, b):
      pltpu.make_async_copy(rows.at[b], acc_sh.at[idx_v.at[pl.ds(j * C, C)]],
                            ssems[b]).wait()

    pltpu.async_copy(idx_hbm.at[pl.ds(base_e, EW)], idx_v, isem)
    for b in range(G):
      gather_start(b, b)

    r0 = s * SPT

    @pl.when(c == 0)
    def _():
      pltpu.sync_copy(src_hbm.at[pl.ds(N_EDGES + r0, SPT)],
                      acc_sh.at[pl.ds(r0, SPT)])

      @pl.when(s == NS - 1)
      def _():
        pltpu.sync_copy(src_hbm.at[pl.ds(N_EDGES + NS * SPT, REM)],
                        acc_sh.at[pl.ds(NS * SPT, REM)])

    @pl.when(c == 1)
    def _():
      pltpu.sync_copy(zeros_hbm.at[pl.ds(0, SPT)],
                      acc_sh.at[pl.ds(r0, SPT)])

      @pl.when(s == NS - 1)
      def _():
        pltpu.sync_copy(zeros_hbm.at[pl.ds(0, REM)],
                        acc_sh.at[pl.ds(NS * SPT, REM)])

    pltpu.make_async_copy(idx_hbm.at[pl.ds(base_e, EW)], idx_v,
                          isem).wait()
    plsc.subcore_barrier()

    for b in range(NB):
      if b >= S:
        scatter_wait(b - S, b - S)
      gather_start(b + G, (b + G) % NB)
      gather_wait(b, b)
      scatter_start(b, b)

    def group(i, carry):
      for b in range(NB):
        j = NB * i + b
        scatter_wait(j - S, (b - S) % NB)
        gather_start(j + G, (b + G) % NB)
        gather_wait(j, b)
        scatter_start(j, b)
      return carry

    lax.fori_loop(1, K // NB - 1, group, 0)

    tail = (K // NB - 1) * NB
    for t in range(tail, K):
      b = t % NB
      scatter_wait(t - S, (b - S) % NB)
      if t + G < K:
        gather_start(t + G, (t + G) % NB)
      gather_wait(t, b)
      scatter_start(t, b)
    for t in range(K - S, K):
      scatter_wait(t, t % NB)

    plsc.subcore_barrier()

    r0 = s * SPT
    pltpu.sync_copy(acc_sh.at[pl.ds(r0, SPT)],
                    out_hbm.at[c, pl.ds(r0, SPT)])

    @pl.when(s == NS - 1)
    def _():
      pltpu.sync_copy(acc_sh.at[pl.ds(NS * SPT, REM)],
                      out_hbm.at[c, pl.ds(NS * SPT, REM)])

  return body(src_emb, dst_ids, zeros)


def _final_add(partials):
  def body(a_ref, o_ref):
    o_ref[...] = a_ref[0] + a_ref[1]

  return pl.pallas_call(
      body,
      grid=(2,),
      in_specs=[pl.BlockSpec((NC, 5000, D), lambda i: (0, i, 0))],
      out_specs=pl.BlockSpec((5000, D), lambda i: (i, 0)),
      out_shape=jax.ShapeDtypeStruct((N_DST, D), jnp.float32),
  )(partials)


ZR = SPT + REM


def kernel(src_emb, src_emb_in, dst_ids):
  del src_emb_in
  zeros = jnp.zeros((ZR, D), jnp.float32)
  partials = _sc_partials(src_emb, dst_ids.astype(jnp.int32), zeros)
  return _final_add(partials)
